# baseline (speedup 1.0000x reference)
import functools

import jax
import jax.numpy as jnp
from jax import lax
from jax.experimental import pallas as pl
from jax.experimental.pallas import tpu as pltpu
from jax.experimental.pallas import tpu_sc as plsc

BATCH = 16384
D = 16
PACK = 128 // D

_info = plsc.get_sparse_core_info()
NC = _info.num_cores
NS = _info.num_subcores
L = _info.num_lanes
NW = NC * NS
BPW = BATCH // NW
CHUNK = 128
NCHUNK = BPW // CHUNK
VECS = CHUNK // L

_mesh = plsc.VectorSubcoreMesh(core_axis_name="c", subcore_axis_name="s")


@functools.partial(
    pl.kernel,
    mesh=_mesh,
    out_type=jax.ShapeDtypeStruct((BATCH,), jnp.float32),
    scratch_types=[
        pltpu.VMEM((NCHUNK, CHUNK), jnp.int32),
        pltpu.VMEM((NCHUNK, CHUNK), jnp.int32),
        pltpu.VMEM((NCHUNK, CHUNK), jnp.int32),
        pltpu.VMEM((NCHUNK, CHUNK), jnp.int32),
        pltpu.VMEM((2, CHUNK, PACK * D), jnp.float32),
        pltpu.VMEM((2, CHUNK, PACK * D), jnp.float32),
        pltpu.VMEM((BPW,), jnp.float32),
        pltpu.SemaphoreType.DMA,
        pltpu.SemaphoreType.DMA,
    ],
    compiler_params=pltpu.CompilerParams(needs_layout_passes=False),
)
def _pmf_sc(uidx_hbm, iidx_hbm, wu_hbm, wi_hbm, out_hbm,
            uidx_v, iidx_v, ublk_v, iblk_v, urows_v, irows_v, out_v,
            usem, isem):
    wid = lax.axis_index("s") * NC + lax.axis_index("c")
    base_row = wid * NCHUNK

    pltpu.sync_copy(uidx_hbm.at[pl.ds(base_row, NCHUNK)], uidx_v)
    pltpu.sync_copy(iidx_hbm.at[pl.ds(base_row, NCHUNK)], iidx_v)

    for c in range(NCHUNK):
        for j in range(VECS):
            s = pl.ds(j * L, L)
            ublk_v[c, s] = lax.shift_right_logical(uidx_v[c, s], 3)
            iblk_v[c, s] = lax.shift_right_logical(iidx_v[c, s], 3)

    def start_chunk(c):
        b = c % 2
        ucp = pltpu.async_copy(wu_hbm.at[ublk_v.at[c]], urows_v.at[b], usem)
        icp = pltpu.async_copy(wi_hbm.at[iblk_v.at[c]], irows_v.at[b], isem)
        return ucp, icp

    lane = lax.iota(jnp.int32, L)
    inflight = start_chunk(0)

    for c in range(NCHUNK):
        ucp, icp = inflight
        ucp.wait()
        icp.wait()
        if c + 1 < NCHUNK:
            inflight = start_chunk(c + 1)
        b = c % 2
        ub = urows_v.at[b]
        ib = irows_v.at[b]

        def group_body(g, _, c=c, ub=ub, ib=ib):
            rows = g * L + lane
            s = pl.ds(g * L, L)
            ucol0 = (uidx_v[c, s] & 7) * D
            icol0 = (iidx_v[c, s] & 7) * D
            acc = jnp.zeros((L,), jnp.float32)
            for d in range(D):
                uc = plsc.load_gather(ub, [rows, ucol0 + d])
                ic = plsc.load_gather(ib, [rows, icol0 + d])
                acc = acc + uc * ic
            out_v[pl.ds(c * CHUNK + g * L, L)] = acc
            return 0

        lax.fori_loop(0, VECS, group_body, 0)

    pltpu.sync_copy(out_v, out_hbm.at[pl.ds(wid * BPW, BPW)])


def kernel(uesr_indices, item_indices, w_user, w_item):
    uidx = uesr_indices.astype(jnp.int32).reshape(NW * NCHUNK, CHUNK)
    iidx = item_indices.astype(jnp.int32).reshape(NW * NCHUNK, CHUNK)
    wu = w_user.reshape(-1, PACK * D)
    wi = w_item.reshape(-1, PACK * D)
    return _pmf_sc(uidx, iidx, wu, wi)
